# baseline (speedup 1.0000x reference)
import jax
import jax.numpy as jnp
from jax.experimental import pallas as pl
from jax.experimental.pallas import tpu as pltpu

_F, _N, _B, _C = 2, 2, 16384, 1000
_M = 0.3
_T = 0.1
_LAMBDA_P = 0.1
_RHO = 0.01

_CM = 200
_NJ = _C // _CM
_NS = _F * _N
_NEG = -1e10


def _loss_body(y_ref, lab_ref, pc_ref, out_ref, fy_scr, mx_scr):
    s = pl.program_id(0)
    j = pl.program_id(1)

    @pl.when((s == 0) & (j == 0))
    def _init():
        out_ref[0, 0] = 0.0

    @pl.when(j == 0)
    def _reset():
        fy_scr[...] = jnp.zeros((1, _B), jnp.float32)
        mx_scr[...] = jnp.full((1, _B), _NEG, jnp.float32)

    yb = y_ref[0, 0]
    lab = lab_ref[...]
    crow = jax.lax.broadcasted_iota(jnp.int32, (_CM, _B), 0) + j * _CM
    mask = crow == lab
    fy_scr[...] += jnp.sum(jnp.where(mask, yb, 0.0), axis=0, keepdims=True)
    blk_mx = jnp.max(jnp.where(mask, _NEG, yb), axis=0, keepdims=True)
    mx_scr[...] = jnp.maximum(mx_scr[...], blk_mx)

    @pl.when(j == _NJ - 1)
    def _slice_done():
        l = jnp.maximum(_M + _T - fy_scr[...], 0.0) + jnp.maximum(
            _M + mx_scr[...], 0.0
        )
        out_ref[0, 0] += jnp.sum(l) * (1.0 / (_NS * _B))

    @pl.when((s == _NS - 1) & (j == _NJ - 1))
    def _fini():
        pc = pc_ref[0, 0]
        out_ref[0, 0] += _LAMBDA_P * pc + (_RHO / 2.0) * pc * pc


def kernel(y, label, power_ratio, power_consumption):
    del power_ratio
    yt = jnp.transpose(y, (0, 1, 3, 2))
    lab2 = label[None, :]
    pc = power_consumption.reshape(1, 1)

    out = pl.pallas_call(
        _loss_body,
        grid=(_NS, _NJ),
        in_specs=[
            pl.BlockSpec((1, 1, _CM, _B), lambda s, j: (s // _N, s % _N, j, 0)),
            pl.BlockSpec((1, _B), lambda s, j: (0, 0)),
            pl.BlockSpec(memory_space=pltpu.SMEM),
        ],
        out_specs=pl.BlockSpec(memory_space=pltpu.SMEM),
        out_shape=jax.ShapeDtypeStruct((1, 1), jnp.float32),
        scratch_shapes=[
            pltpu.VMEM((1, _B), jnp.float32),
            pltpu.VMEM((1, _B), jnp.float32),
        ],
        compiler_params=pltpu.CompilerParams(
            dimension_semantics=("arbitrary", "arbitrary"),
        ),
    )(yt, lab2, pc)
    return out.reshape(1)
